# baseline (speedup 1.0000x reference)
import functools

import jax
import jax.numpy as jnp
from jax import lax
from jax.experimental import pallas as pl
from jax.experimental.pallas import tpu as pltpu
from jax.experimental.pallas import tpu_sc as plsc

B = 16384
D = 64
NC = 2
NS = 16
NW = NC * NS
BPW = B // NW
CH = 128
NCH = BPW // CH
L = 16


def _bpr_body(user_i, item_i_i, item_j_i, utab, itab, out_i, out_j,
              uidx, iidx, jidx, urows, virows, vjrows, oi, oj, sem):
    wid = lax.axis_index("s") * NC + lax.axis_index("c")
    base = wid * NCH

    pltpu.sync_copy(user_i.at[pl.ds(base, NCH)], uidx)
    pltpu.sync_copy(item_i_i.at[pl.ds(base, NCH)], iidx)
    pltpu.sync_copy(item_j_i.at[pl.ds(base, NCH)], jidx)

    cps = []
    for k in range(NCH):
        cps.append(pltpu.async_copy(
            utab.at[uidx.at[k]], urows.at[pl.ds(k * CH, CH)], sem))
        cps.append(pltpu.async_copy(
            itab.at[iidx.at[k]], virows.at[pl.ds(k * CH, CH)], sem))
        cps.append(pltpu.async_copy(
            itab.at[jidx.at[k]], vjrows.at[pl.ds(k * CH, CH)], sem))
    for cp in cps:
        cp.wait()

    lane = lax.iota(jnp.int32, L)
    last = lane == (L - 1)

    def row(r, carry):
        acc_i = jnp.zeros((L,), jnp.float32)
        acc_j = jnp.zeros((L,), jnp.float32)
        for cc in range(D // L):
            u = urows[r, pl.ds(cc * L, L)]
            vi = virows[r, pl.ds(cc * L, L)]
            vj = vjrows[r, pl.ds(cc * L, L)]
            acc_i = acc_i + u * vi
            acc_j = acc_j + u * vj
        ridx = jnp.full((L,), r, jnp.int32)
        plsc.store_scatter(oi, [ridx], plsc.cumsum(acc_i), mask=last)
        plsc.store_scatter(oj, [ridx], plsc.cumsum(acc_j), mask=last)
        return carry

    lax.fori_loop(0, BPW, row, 0)

    ob = wid * BPW
    pltpu.sync_copy(oi, out_i.at[pl.ds(ob, BPW)])
    pltpu.sync_copy(oj, out_j.at[pl.ds(ob, BPW)])


def kernel(user, item_i, item_j, embed_user_weight, embed_item_weight):
    mesh = plsc.VectorSubcoreMesh(core_axis_name="c", subcore_axis_name="s")
    run = pl.kernel(
        _bpr_body,
        mesh=mesh,
        compiler_params=pltpu.CompilerParams(
            needs_layout_passes=False, use_tc_tiling_on_sc=False),
        out_type=(
            jax.ShapeDtypeStruct((B,), jnp.float32),
            jax.ShapeDtypeStruct((B,), jnp.float32),
        ),
        scratch_types=[
            pltpu.VMEM((NCH, CH), jnp.int32),
            pltpu.VMEM((NCH, CH), jnp.int32),
            pltpu.VMEM((NCH, CH), jnp.int32),
            pltpu.VMEM((BPW, D), jnp.float32),
            pltpu.VMEM((BPW, D), jnp.float32),
            pltpu.VMEM((BPW, D), jnp.float32),
            pltpu.VMEM((BPW,), jnp.float32),
            pltpu.VMEM((BPW,), jnp.float32),
            pltpu.SemaphoreType.DMA,
        ],
    )
    u2 = user.astype(jnp.int32).reshape(NW * NCH, CH)
    i2 = item_i.astype(jnp.int32).reshape(NW * NCH, CH)
    j2 = item_j.astype(jnp.int32).reshape(NW * NCH, CH)
    return run(u2, i2, j2, embed_user_weight, embed_item_weight)


# speedup vs baseline: 1.0868x; 1.0868x over previous
import functools

import jax
import jax.numpy as jnp
from jax import lax
from jax.experimental import pallas as pl
from jax.experimental.pallas import tpu as pltpu
from jax.experimental.pallas import tpu_sc as plsc

B = 16384
D = 64
DP = 128
NC = 2
NS = 16
NW = NC * NS
BPW = B // NW
CH = 128
NCH = BPW // CH
L = 16


def _bpr_body(user_i, item_i_i, item_j_i, utab, itab, out_i, out_j,
              uidx, iidx, jidx, urows, virows, vjrows, oi, oj, sem):
    wid = lax.axis_index("s") * NC + lax.axis_index("c")
    base = wid * NCH

    pltpu.sync_copy(user_i.at[pl.ds(base, NCH)], uidx)
    pltpu.sync_copy(item_i_i.at[pl.ds(base, NCH)], iidx)
    pltpu.sync_copy(item_j_i.at[pl.ds(base, NCH)], jidx)

    lane = lax.iota(jnp.int32, L)
    last = lane == (L - 1)

    for k in range(NCH):
        cps = [
            pltpu.async_copy(utab.at[uidx.at[k]], urows, sem),
            pltpu.async_copy(itab.at[iidx.at[k]], virows, sem),
            pltpu.async_copy(itab.at[jidx.at[k]], vjrows, sem),
        ]
        for cp in cps:
            cp.wait()

        def row(r, carry):
            acc_i = jnp.zeros((L,), jnp.float32)
            acc_j = jnp.zeros((L,), jnp.float32)
            for cc in range(D // L):
                u = urows[r, pl.ds(cc * L, L)]
                vi = virows[r, pl.ds(cc * L, L)]
                vj = vjrows[r, pl.ds(cc * L, L)]
                acc_i = acc_i + u * vi
                acc_j = acc_j + u * vj
            gidx = jnp.full((L,), k * CH + r, jnp.int32)
            plsc.store_scatter(oi, [gidx], plsc.cumsum(acc_i), mask=last)
            plsc.store_scatter(oj, [gidx], plsc.cumsum(acc_j), mask=last)
            return carry

        lax.fori_loop(0, CH, row, 0)

    ob = wid * BPW
    pltpu.sync_copy(oi, out_i.at[pl.ds(ob, BPW)])
    pltpu.sync_copy(oj, out_j.at[pl.ds(ob, BPW)])


def kernel(user, item_i, item_j, embed_user_weight, embed_item_weight):
    mesh = plsc.VectorSubcoreMesh(core_axis_name="c", subcore_axis_name="s")
    run = pl.kernel(
        _bpr_body,
        mesh=mesh,
        compiler_params=pltpu.CompilerParams(
            needs_layout_passes=False, use_tc_tiling_on_sc=True),
        out_type=(
            jax.ShapeDtypeStruct((B,), jnp.float32),
            jax.ShapeDtypeStruct((B,), jnp.float32),
        ),
        scratch_types=[
            pltpu.VMEM((NCH, CH), jnp.int32),
            pltpu.VMEM((NCH, CH), jnp.int32),
            pltpu.VMEM((NCH, CH), jnp.int32),
            pltpu.VMEM((CH, DP), jnp.float32),
            pltpu.VMEM((CH, DP), jnp.float32),
            pltpu.VMEM((CH, DP), jnp.float32),
            pltpu.VMEM((BPW,), jnp.float32),
            pltpu.VMEM((BPW,), jnp.float32),
            pltpu.SemaphoreType.DMA,
        ],
    )
    u2 = user.astype(jnp.int32).reshape(NW * NCH, CH)
    i2 = item_i.astype(jnp.int32).reshape(NW * NCH, CH)
    j2 = item_j.astype(jnp.int32).reshape(NW * NCH, CH)
    utab2 = jnp.pad(embed_user_weight, ((0, 0), (0, DP - D)))
    itab2 = jnp.pad(embed_item_weight, ((0, 0), (0, DP - D)))
    return run(u2, i2, j2, utab2, itab2)
